# Initial kernel scaffold; baseline (speedup 1.0000x reference)
import jax
import jax.numpy as jnp
from jax.experimental import pallas as pl


def kernel(x, coords, conv1_w, conv2_w, bn1_g, bn1_b, bn2_g, bn2_b, k):
    raise NotImplementedError("write your pallas kernel here")


# speedup vs baseline: 6.1169x; 6.1169x over previous
import functools

import jax
import jax.numpy as jnp
from jax import lax
from jax.experimental import pallas as pl
from jax.experimental.pallas import tpu as pltpu
from jax.experimental.pallas import tpu_sc as plsc

B, N, D, S, K, C = 8, 2048, 128, 512, 32, 256
M = B * S
R = M * K
TS = 256
TR = 1024
NML = float(M * K)
EPS = 1e-5


def _topk_body(a_ref, bm_ref, out_ref):
    b = pl.program_id(0)
    a = a_ref[0]
    bm = bm_ref[0]
    dot = jnp.dot(a, bm, preferred_element_type=jnp.float32)
    ax, ay, az = a[:, 0:1], a[:, 1:2], a[:, 2:3]
    cx, cy, cz = bm[0:1, :], bm[1:2, :], bm[2:3, :]
    nc2 = ax * ax + ay * ay + az * az
    c2 = cx * cx + cy * cy + cz * cz
    d2 = (nc2 - 2.0 * dot) + c2
    col = lax.broadcasted_iota(jnp.int32, (TS, N), 1)
    lane = lax.broadcasted_iota(jnp.int32, (TS, K), 1)
    res = jnp.zeros((TS, K), jnp.int32)
    inf = jnp.float32(jnp.inf)
    for j in range(K):
        vmin = jnp.min(d2, axis=1, keepdims=True)
        cand = jnp.where(d2 == vmin, col, N)
        amin = jnp.min(cand, axis=1, keepdims=True)
        res = jnp.where(lane == j, amin, res)
        d2 = jnp.where(col == amin, inf, d2)
    out_ref[0] = res + b * N


def _topk_indices(coords):
    a = jnp.pad(coords[:, ::4, :], ((0, 0), (0, 0), (0, 5)))
    bm = jnp.pad(jnp.transpose(coords, (0, 2, 1)),
                 ((0, 0), (0, 5), (0, 0)))
    idx = pl.pallas_call(
        _topk_body,
        grid=(B, S // TS),
        in_specs=[
            pl.BlockSpec((1, TS, 8), lambda b, t: (b, t, 0)),
            pl.BlockSpec((1, 8, N), lambda b, t: (b, 0, 0)),
        ],
        out_specs=pl.BlockSpec((1, TS, K), lambda b, t: (b, t, 0)),
        out_shape=jax.ShapeDtypeStruct((B, S, K), jnp.int32),
        compiler_params=pltpu.CompilerParams(
            dimension_semantics=("arbitrary", "arbitrary")),
    )(a, bm)
    return idx.reshape(R)


IDX_COLS = 128
ROWS_PER_W = R // 32
CHUNKS = ROWS_PER_W // IDX_COLS


def _sc_gather(table, idx2d):
    mesh = plsc.VectorSubcoreMesh(core_axis_name="c", subcore_axis_name="s")

    @functools.partial(
        pl.kernel,
        mesh=mesh,
        out_type=jax.ShapeDtypeStruct((R, D), jnp.float32),
        scratch_types=[
            pltpu.VMEM((CHUNKS, IDX_COLS), jnp.int32),
            pltpu.VMEM((IDX_COLS, D), jnp.float32),
            pltpu.SemaphoreType.DMA,
        ],
    )
    def k(table_hbm, idx_hbm, out_hbm, idx_v, buf_v, sem):
        wid = lax.axis_index("s") * 2 + lax.axis_index("c")
        pltpu.sync_copy(idx_hbm.at[pl.ds(wid * CHUNKS, CHUNKS)], idx_v)

        def body(i, _):
            pltpu.async_copy(table_hbm.at[idx_v.at[i]], buf_v, sem).wait()
            pltpu.sync_copy(
                buf_v, out_hbm.at[pl.ds(wid * ROWS_PER_W + i * IDX_COLS,
                                        IDX_COLS)])
            return 0

        lax.fori_loop(0, CHUNKS, body, 0)

    return k(table, idx2d)


def _c1_body(g_ref, c_ref, w1a_ref, w1d_ref, h1_ref, st_ref):
    h = jnp.dot(g_ref[...], w1a_ref[...], preferred_element_type=jnp.float32)
    v = jnp.dot(c_ref[...], w1d_ref[...], preferred_element_type=jnp.float32)
    h = (h.reshape(TR // K, K, C) + v[:, None, :]).reshape(TR, C)
    h1_ref[...] = h

    @pl.when(pl.program_id(0) == 0)
    def _():
        st_ref[...] = jnp.zeros((8, C), jnp.float32)

    st_ref[0:1, :] += jnp.sum(h, axis=0, keepdims=True)
    st_ref[1:2, :] += jnp.sum(h * h, axis=0, keepdims=True)


def _c2_body(h1_ref, st1_ref, w2_ref, g1_ref, b1_ref, p_ref, st_ref):
    st = st1_ref[...]
    mean = st[0:1, :] * (1.0 / NML)
    var = st[1:2, :] * (1.0 / NML) - mean * mean
    rstd = 1.0 / jnp.sqrt(var + EPS)
    scale = rstd * g1_ref[...]
    shift = b1_ref[...] - mean * scale
    a = jnp.maximum(h1_ref[...] * scale + shift, 0.0)
    h2 = jnp.dot(a, w2_ref[...], preferred_element_type=jnp.float32)
    p_ref[...] = jnp.max(h2.reshape(TR // K, K, C), axis=1)

    @pl.when(pl.program_id(0) == 0)
    def _():
        st_ref[...] = jnp.zeros((8, C), jnp.float32)

    st_ref[0:1, :] += jnp.sum(h2, axis=0, keepdims=True)
    st_ref[1:2, :] += jnp.sum(h2 * h2, axis=0, keepdims=True)


def _c3_body(p_ref, st2_ref, g2_ref, b2_ref, out_ref):
    st = st2_ref[...]
    mean = st[0:1, :] * (1.0 / NML)
    var = st[1:2, :] * (1.0 / NML) - mean * mean
    rstd = 1.0 / jnp.sqrt(var + EPS)
    scale = rstd * g2_ref[...]
    shift = b2_ref[...] - mean * scale
    out_ref[...] = jnp.maximum(p_ref[...] * scale + shift, 0.0)


def _conv_pipeline(G, centers, w1aT, w1dT, w2T, bn1_g, bn1_b, bn2_g, bn2_b):
    nsteps = R // TR
    h1, st1 = pl.pallas_call(
        _c1_body,
        grid=(nsteps,),
        in_specs=[
            pl.BlockSpec((TR, D), lambda t: (t, 0)),
            pl.BlockSpec((TR // K, D), lambda t: (t, 0)),
            pl.BlockSpec((D, C), lambda t: (0, 0)),
            pl.BlockSpec((D, C), lambda t: (0, 0)),
        ],
        out_specs=[
            pl.BlockSpec((TR, C), lambda t: (t, 0)),
            pl.BlockSpec((8, C), lambda t: (0, 0)),
        ],
        out_shape=[
            jax.ShapeDtypeStruct((R, C), jnp.float32),
            jax.ShapeDtypeStruct((8, C), jnp.float32),
        ],
        compiler_params=pltpu.CompilerParams(
            dimension_semantics=("arbitrary",)),
    )(G, centers, w1aT, w1dT)

    p, st2 = pl.pallas_call(
        _c2_body,
        grid=(nsteps,),
        in_specs=[
            pl.BlockSpec((TR, C), lambda t: (t, 0)),
            pl.BlockSpec((8, C), lambda t: (0, 0)),
            pl.BlockSpec((C, C), lambda t: (0, 0)),
            pl.BlockSpec((1, C), lambda t: (0, 0)),
            pl.BlockSpec((1, C), lambda t: (0, 0)),
        ],
        out_specs=[
            pl.BlockSpec((TR // K, C), lambda t: (t, 0)),
            pl.BlockSpec((8, C), lambda t: (0, 0)),
        ],
        out_shape=[
            jax.ShapeDtypeStruct((M, C), jnp.float32),
            jax.ShapeDtypeStruct((8, C), jnp.float32),
        ],
        compiler_params=pltpu.CompilerParams(
            dimension_semantics=("arbitrary",)),
    )(h1, st1, w2T, bn1_g.reshape(1, C), bn1_b.reshape(1, C))

    out = pl.pallas_call(
        _c3_body,
        grid=(4,),
        in_specs=[
            pl.BlockSpec((M // 4, C), lambda t: (t, 0)),
            pl.BlockSpec((8, C), lambda t: (0, 0)),
            pl.BlockSpec((1, C), lambda t: (0, 0)),
            pl.BlockSpec((1, C), lambda t: (0, 0)),
        ],
        out_specs=pl.BlockSpec((M // 4, C), lambda t: (t, 0)),
        out_shape=jax.ShapeDtypeStruct((M, C), jnp.float32),
        compiler_params=pltpu.CompilerParams(
            dimension_semantics=("arbitrary",)),
    )(p, st2, bn2_g.reshape(1, C), bn2_b.reshape(1, C))
    return out


def kernel(x, coords, conv1_w, conv2_w, bn1_g, bn1_b, bn2_g, bn2_b, k):
    del k
    idx = _topk_indices(coords)
    feats2d = jnp.transpose(x, (0, 2, 1)).reshape(B * N, D)
    G = _sc_gather(feats2d, idx.reshape(R // IDX_COLS, IDX_COLS))
    centers = feats2d.reshape(B, N, D)[:, ::4, :].reshape(M, D)
    w1a = conv1_w[:, :D]
    w1d = conv1_w[:, D:] - w1a
    out = _conv_pipeline(G, centers, w1a.T, w1d.T, conv2_w.T,
                         bn1_g, bn1_b, bn2_g, bn2_b)
    h = out.reshape(B, S, C).transpose(0, 2, 1)
    return (coords, h)


# speedup vs baseline: 7.2294x; 1.1819x over previous
import functools

import jax
import jax.numpy as jnp
from jax import lax
from jax.experimental import pallas as pl
from jax.experimental.pallas import tpu as pltpu
from jax.experimental.pallas import tpu_sc as plsc

B, N, D, S, K, C = 8, 2048, 128, 512, 32, 256
M = B * S
R = M * K
TS = 256
TR = 1024
NML = float(M * K)
EPS = 1e-5


def _topk_body(a_ref, bm_ref, out_ref):
    b = pl.program_id(0)
    a = a_ref[0]
    bm = bm_ref[0]
    dot = jnp.dot(a, bm, preferred_element_type=jnp.float32)
    ax, ay, az = a[:, 0:1], a[:, 1:2], a[:, 2:3]
    cx, cy, cz = bm[0:1, :], bm[1:2, :], bm[2:3, :]
    nc2 = ax * ax + ay * ay + az * az
    c2 = cx * cx + cy * cy + cz * cz
    d2 = (nc2 - 2.0 * dot) + c2
    col = lax.broadcasted_iota(jnp.int32, (TS, N), 1).astype(jnp.float32)
    lane = lax.broadcasted_iota(jnp.int32, (TS, K), 1).astype(jnp.float32)
    res = jnp.zeros((TS, K), jnp.float32)
    inf = jnp.float32(jnp.inf)
    fn = jnp.float32(N)
    for j in range(K):
        vmin = jnp.min(d2, axis=1, keepdims=True)
        cand = jnp.where(d2 == vmin, col, fn)
        amin = jnp.min(cand, axis=1, keepdims=True)
        res = jnp.where(lane == j, amin, res)
        d2 = jnp.where(col == amin, inf, d2)
    out_ref[0] = res.astype(jnp.int32) + b * N


def _topk_indices(coords):
    a = jnp.pad(coords[:, ::4, :], ((0, 0), (0, 0), (0, 5)))
    bm = jnp.pad(jnp.transpose(coords, (0, 2, 1)),
                 ((0, 0), (0, 5), (0, 0)))
    idx = pl.pallas_call(
        _topk_body,
        grid=(B, S // TS),
        in_specs=[
            pl.BlockSpec((1, TS, 8), lambda b, t: (b, t, 0)),
            pl.BlockSpec((1, 8, N), lambda b, t: (b, 0, 0)),
        ],
        out_specs=pl.BlockSpec((1, TS, K), lambda b, t: (b, t, 0)),
        out_shape=jax.ShapeDtypeStruct((B, S, K), jnp.int32),
        compiler_params=pltpu.CompilerParams(
            dimension_semantics=("arbitrary", "arbitrary")),
    )(a, bm)
    return idx.reshape(R)


IDX_COLS = 128
ROWS_PER_W = R // 32
CHUNKS = ROWS_PER_W // IDX_COLS


def _sc_gather(table, idx2d):
    mesh = plsc.VectorSubcoreMesh(core_axis_name="c", subcore_axis_name="s")

    @functools.partial(
        pl.kernel,
        mesh=mesh,
        out_type=jax.ShapeDtypeStruct((R, D), jnp.float32),
        scratch_types=[
            pltpu.VMEM((CHUNKS, IDX_COLS), jnp.int32),
            pltpu.VMEM((IDX_COLS, D), jnp.float32),
            pltpu.SemaphoreType.DMA,
        ],
    )
    def k(table_hbm, idx_hbm, out_hbm, idx_v, buf_v, sem):
        wid = lax.axis_index("s") * 2 + lax.axis_index("c")
        pltpu.sync_copy(idx_hbm.at[pl.ds(wid * CHUNKS, CHUNKS)], idx_v)

        def body(i, _):
            pltpu.async_copy(table_hbm.at[idx_v.at[i]], buf_v, sem).wait()
            pltpu.sync_copy(
                buf_v, out_hbm.at[pl.ds(wid * ROWS_PER_W + i * IDX_COLS,
                                        IDX_COLS)])
            return 0

        lax.fori_loop(0, CHUNKS, body, 0)

    return k(table, idx2d)


def _c1_body(g_ref, c_ref, w1a_ref, w1d_ref, h1_ref, st_ref):
    bf = jnp.bfloat16
    h = jnp.dot(g_ref[...].astype(bf), w1a_ref[...].astype(bf),
                preferred_element_type=jnp.float32)
    v = jnp.dot(c_ref[...].astype(bf), w1d_ref[...].astype(bf),
                preferred_element_type=jnp.float32)
    h = (h.reshape(TR // K, K, C) + v[:, None, :]).reshape(TR, C)
    h1_ref[...] = h.astype(jnp.bfloat16)

    @pl.when(pl.program_id(0) == 0)
    def _():
        st_ref[...] = jnp.zeros((8, C), jnp.float32)

    st_ref[0:1, :] += jnp.sum(h, axis=0, keepdims=True)
    st_ref[1:2, :] += jnp.sum(h * h, axis=0, keepdims=True)


def _c2_body(h1_ref, st1_ref, w2_ref, g1_ref, b1_ref, p_ref, st_ref):
    st = st1_ref[...]
    mean = st[0:1, :] * (1.0 / NML)
    var = st[1:2, :] * (1.0 / NML) - mean * mean
    rstd = 1.0 / jnp.sqrt(var + EPS)
    scale = rstd * g1_ref[...]
    shift = b1_ref[...] - mean * scale
    a = jnp.maximum(h1_ref[...].astype(jnp.float32) * scale + shift, 0.0)
    h2 = jnp.dot(a.astype(jnp.bfloat16), w2_ref[...].astype(jnp.bfloat16),
                 preferred_element_type=jnp.float32)
    p_ref[...] = jnp.max(h2.reshape(TR // K, K, C), axis=1)

    @pl.when(pl.program_id(0) == 0)
    def _():
        st_ref[...] = jnp.zeros((8, C), jnp.float32)

    st_ref[0:1, :] += jnp.sum(h2, axis=0, keepdims=True)
    st_ref[1:2, :] += jnp.sum(h2 * h2, axis=0, keepdims=True)


def _c3_body(p_ref, st2_ref, g2_ref, b2_ref, out_ref):
    st = st2_ref[...]
    mean = st[0:1, :] * (1.0 / NML)
    var = st[1:2, :] * (1.0 / NML) - mean * mean
    rstd = 1.0 / jnp.sqrt(var + EPS)
    scale = rstd * g2_ref[...]
    shift = b2_ref[...] - mean * scale
    out_ref[...] = jnp.maximum(p_ref[...] * scale + shift, 0.0)


def _conv_pipeline(G, centers, w1aT, w1dT, w2T, bn1_g, bn1_b, bn2_g, bn2_b):
    nsteps = R // TR
    h1, st1 = pl.pallas_call(
        _c1_body,
        grid=(nsteps,),
        in_specs=[
            pl.BlockSpec((TR, D), lambda t: (t, 0)),
            pl.BlockSpec((TR // K, D), lambda t: (t, 0)),
            pl.BlockSpec((D, C), lambda t: (0, 0)),
            pl.BlockSpec((D, C), lambda t: (0, 0)),
        ],
        out_specs=[
            pl.BlockSpec((TR, C), lambda t: (t, 0)),
            pl.BlockSpec((8, C), lambda t: (0, 0)),
        ],
        out_shape=[
            jax.ShapeDtypeStruct((R, C), jnp.bfloat16),
            jax.ShapeDtypeStruct((8, C), jnp.float32),
        ],
        compiler_params=pltpu.CompilerParams(
            dimension_semantics=("arbitrary",)),
    )(G, centers, w1aT, w1dT)

    p, st2 = pl.pallas_call(
        _c2_body,
        grid=(nsteps,),
        in_specs=[
            pl.BlockSpec((TR, C), lambda t: (t, 0)),
            pl.BlockSpec((8, C), lambda t: (0, 0)),
            pl.BlockSpec((C, C), lambda t: (0, 0)),
            pl.BlockSpec((1, C), lambda t: (0, 0)),
            pl.BlockSpec((1, C), lambda t: (0, 0)),
        ],
        out_specs=[
            pl.BlockSpec((TR // K, C), lambda t: (t, 0)),
            pl.BlockSpec((8, C), lambda t: (0, 0)),
        ],
        out_shape=[
            jax.ShapeDtypeStruct((M, C), jnp.float32),
            jax.ShapeDtypeStruct((8, C), jnp.float32),
        ],
        compiler_params=pltpu.CompilerParams(
            dimension_semantics=("arbitrary",)),
    )(h1, st1, w2T, bn1_g.reshape(1, C), bn1_b.reshape(1, C))

    out = pl.pallas_call(
        _c3_body,
        grid=(4,),
        in_specs=[
            pl.BlockSpec((M // 4, C), lambda t: (t, 0)),
            pl.BlockSpec((8, C), lambda t: (0, 0)),
            pl.BlockSpec((1, C), lambda t: (0, 0)),
            pl.BlockSpec((1, C), lambda t: (0, 0)),
        ],
        out_specs=pl.BlockSpec((M // 4, C), lambda t: (t, 0)),
        out_shape=jax.ShapeDtypeStruct((M, C), jnp.float32),
        compiler_params=pltpu.CompilerParams(
            dimension_semantics=("arbitrary",)),
    )(p, st2, bn2_g.reshape(1, C), bn2_b.reshape(1, C))
    return out


def kernel(x, coords, conv1_w, conv2_w, bn1_g, bn1_b, bn2_g, bn2_b, k):
    del k
    idx = _topk_indices(coords)
    feats2d = jnp.transpose(x, (0, 2, 1)).reshape(B * N, D)
    G = _sc_gather(feats2d, idx.reshape(R // IDX_COLS, IDX_COLS))
    centers = feats2d.reshape(B, N, D)[:, ::4, :].reshape(M, D)
    w1a = conv1_w[:, :D]
    w1d = conv1_w[:, D:] - w1a
    out = _conv_pipeline(G, centers, w1a.T, w1d.T, conv2_w.T,
                         bn1_g, bn1_b, bn2_g, bn2_b)
    h = out.reshape(B, S, C).transpose(0, 2, 1)
    return (coords, h)


# speedup vs baseline: 7.2309x; 1.0002x over previous
import functools

import jax
import jax.numpy as jnp
from jax import lax
from jax.experimental import pallas as pl
from jax.experimental.pallas import tpu as pltpu
from jax.experimental.pallas import tpu_sc as plsc

B, N, D, S, K, C = 8, 2048, 128, 512, 32, 256
M = B * S
R = M * K
TS = 256
TR = 1024
NML = float(M * K)
EPS = 1e-5


def _topk_body(a_ref, bm_ref, out_ref):
    b = pl.program_id(0)
    a = a_ref[0]
    bm = bm_ref[0]
    dot = jnp.dot(a, bm, preferred_element_type=jnp.float32)
    ax, ay, az = a[:, 0:1], a[:, 1:2], a[:, 2:3]
    cx, cy, cz = bm[0:1, :], bm[1:2, :], bm[2:3, :]
    nc2 = ax * ax + ay * ay + az * az
    c2 = cx * cx + cy * cy + cz * cz
    d2 = (nc2 - 2.0 * dot) + c2
    col = lax.broadcasted_iota(jnp.int32, (TS, N), 1).astype(jnp.float32)
    lane = lax.broadcasted_iota(jnp.int32, (TS, K), 1).astype(jnp.float32)
    res = jnp.zeros((TS, K), jnp.float32)
    inf = jnp.float32(jnp.inf)
    fn = jnp.float32(N)
    for j in range(K):
        vmin = jnp.min(d2, axis=1, keepdims=True)
        cand = jnp.where(d2 == vmin, col, fn)
        amin = jnp.min(cand, axis=1, keepdims=True)
        res = jnp.where(lane == j, amin, res)
        d2 = jnp.where(col == amin, inf, d2)
    out_ref[0] = res.astype(jnp.int32) + b * N


def _topk_indices(coords):
    a = jnp.pad(coords[:, ::4, :], ((0, 0), (0, 0), (0, 5)))
    bm = jnp.pad(jnp.transpose(coords, (0, 2, 1)),
                 ((0, 0), (0, 5), (0, 0)))
    idx = pl.pallas_call(
        _topk_body,
        grid=(B, S // TS),
        in_specs=[
            pl.BlockSpec((1, TS, 8), lambda b, t: (b, t, 0)),
            pl.BlockSpec((1, 8, N), lambda b, t: (b, 0, 0)),
        ],
        out_specs=pl.BlockSpec((1, TS, K), lambda b, t: (b, t, 0)),
        out_shape=jax.ShapeDtypeStruct((B, S, K), jnp.int32),
        compiler_params=pltpu.CompilerParams(
            dimension_semantics=("arbitrary", "arbitrary")),
    )(a, bm)
    return idx.reshape(R)


IDX_COLS = 128
ROWS_PER_W = R // 32
CHUNKS = ROWS_PER_W // IDX_COLS


def _sc_gather(table, idx2d):
    mesh = plsc.VectorSubcoreMesh(core_axis_name="c", subcore_axis_name="s")

    @functools.partial(
        pl.kernel,
        mesh=mesh,
        out_type=jax.ShapeDtypeStruct((R, D), jnp.float32),
        scratch_types=[
            pltpu.VMEM((CHUNKS, IDX_COLS), jnp.int32),
            pltpu.VMEM((IDX_COLS, D), jnp.float32),
            pltpu.VMEM((IDX_COLS, D), jnp.float32),
            pltpu.SemaphoreType.DMA,
            pltpu.SemaphoreType.DMA,
        ],
    )
    def k(table_hbm, idx_hbm, out_hbm, idx_v, buf_a, buf_b, sem_a, sem_b):
        wid = lax.axis_index("s") * 2 + lax.axis_index("c")
        base = wid * ROWS_PER_W
        pltpu.sync_copy(idx_hbm.at[pl.ds(wid * CHUNKS, CHUNKS)], idx_v)

        def body(i, _):
            ca = pltpu.async_copy(table_hbm.at[idx_v.at[2 * i]], buf_a, sem_a)
            cb = pltpu.async_copy(table_hbm.at[idx_v.at[2 * i + 1]], buf_b,
                                  sem_b)
            ca.wait()
            pltpu.sync_copy(
                buf_a, out_hbm.at[pl.ds(base + 2 * i * IDX_COLS, IDX_COLS)])
            cb.wait()
            pltpu.sync_copy(
                buf_b, out_hbm.at[pl.ds(base + (2 * i + 1) * IDX_COLS,
                                        IDX_COLS)])
            return 0

        lax.fori_loop(0, CHUNKS // 2, body, 0)

    return k(table, idx2d)


def _c1_body(g_ref, c_ref, w1a_ref, w1d_ref, h1_ref, st_ref):
    bf = jnp.bfloat16
    h = jnp.dot(g_ref[...].astype(bf), w1a_ref[...].astype(bf),
                preferred_element_type=jnp.float32)
    v = jnp.dot(c_ref[...].astype(bf), w1d_ref[...].astype(bf),
                preferred_element_type=jnp.float32)
    h = (h.reshape(TR // K, K, C) + v[:, None, :]).reshape(TR, C)
    h1_ref[...] = h.astype(jnp.bfloat16)

    @pl.when(pl.program_id(0) == 0)
    def _():
        st_ref[...] = jnp.zeros((8, C), jnp.float32)

    st_ref[0:1, :] += jnp.sum(h, axis=0, keepdims=True)
    st_ref[1:2, :] += jnp.sum(h * h, axis=0, keepdims=True)


def _c2_body(h1_ref, st1_ref, w2_ref, g1_ref, b1_ref, p_ref, st_ref):
    st = st1_ref[...]
    mean = st[0:1, :] * (1.0 / NML)
    var = st[1:2, :] * (1.0 / NML) - mean * mean
    rstd = 1.0 / jnp.sqrt(var + EPS)
    scale = rstd * g1_ref[...]
    shift = b1_ref[...] - mean * scale
    a = jnp.maximum(h1_ref[...].astype(jnp.float32) * scale + shift, 0.0)
    h2 = jnp.dot(a.astype(jnp.bfloat16), w2_ref[...].astype(jnp.bfloat16),
                 preferred_element_type=jnp.float32)
    p_ref[...] = jnp.max(h2.reshape(TR // K, K, C), axis=1)

    @pl.when(pl.program_id(0) == 0)
    def _():
        st_ref[...] = jnp.zeros((8, C), jnp.float32)

    st_ref[0:1, :] += jnp.sum(h2, axis=0, keepdims=True)
    st_ref[1:2, :] += jnp.sum(h2 * h2, axis=0, keepdims=True)


def _c3_body(p_ref, st2_ref, g2_ref, b2_ref, out_ref):
    st = st2_ref[...]
    mean = st[0:1, :] * (1.0 / NML)
    var = st[1:2, :] * (1.0 / NML) - mean * mean
    rstd = 1.0 / jnp.sqrt(var + EPS)
    scale = rstd * g2_ref[...]
    shift = b2_ref[...] - mean * scale
    out_ref[...] = jnp.maximum(p_ref[...] * scale + shift, 0.0)


def _conv_pipeline(G, centers, w1aT, w1dT, w2T, bn1_g, bn1_b, bn2_g, bn2_b):
    nsteps = R // TR
    h1, st1 = pl.pallas_call(
        _c1_body,
        grid=(nsteps,),
        in_specs=[
            pl.BlockSpec((TR, D), lambda t: (t, 0)),
            pl.BlockSpec((TR // K, D), lambda t: (t, 0)),
            pl.BlockSpec((D, C), lambda t: (0, 0)),
            pl.BlockSpec((D, C), lambda t: (0, 0)),
        ],
        out_specs=[
            pl.BlockSpec((TR, C), lambda t: (t, 0)),
            pl.BlockSpec((8, C), lambda t: (0, 0)),
        ],
        out_shape=[
            jax.ShapeDtypeStruct((R, C), jnp.bfloat16),
            jax.ShapeDtypeStruct((8, C), jnp.float32),
        ],
        compiler_params=pltpu.CompilerParams(
            dimension_semantics=("arbitrary",)),
    )(G, centers, w1aT, w1dT)

    p, st2 = pl.pallas_call(
        _c2_body,
        grid=(nsteps,),
        in_specs=[
            pl.BlockSpec((TR, C), lambda t: (t, 0)),
            pl.BlockSpec((8, C), lambda t: (0, 0)),
            pl.BlockSpec((C, C), lambda t: (0, 0)),
            pl.BlockSpec((1, C), lambda t: (0, 0)),
            pl.BlockSpec((1, C), lambda t: (0, 0)),
        ],
        out_specs=[
            pl.BlockSpec((TR // K, C), lambda t: (t, 0)),
            pl.BlockSpec((8, C), lambda t: (0, 0)),
        ],
        out_shape=[
            jax.ShapeDtypeStruct((M, C), jnp.float32),
            jax.ShapeDtypeStruct((8, C), jnp.float32),
        ],
        compiler_params=pltpu.CompilerParams(
            dimension_semantics=("arbitrary",)),
    )(h1, st1, w2T, bn1_g.reshape(1, C), bn1_b.reshape(1, C))

    out = pl.pallas_call(
        _c3_body,
        grid=(4,),
        in_specs=[
            pl.BlockSpec((M // 4, C), lambda t: (t, 0)),
            pl.BlockSpec((8, C), lambda t: (0, 0)),
            pl.BlockSpec((1, C), lambda t: (0, 0)),
            pl.BlockSpec((1, C), lambda t: (0, 0)),
        ],
        out_specs=pl.BlockSpec((M // 4, C), lambda t: (t, 0)),
        out_shape=jax.ShapeDtypeStruct((M, C), jnp.float32),
        compiler_params=pltpu.CompilerParams(
            dimension_semantics=("arbitrary",)),
    )(p, st2, bn2_g.reshape(1, C), bn2_b.reshape(1, C))
    return out


def kernel(x, coords, conv1_w, conv2_w, bn1_g, bn1_b, bn2_g, bn2_b, k):
    del k
    idx = _topk_indices(coords)
    feats2d = jnp.transpose(x, (0, 2, 1)).reshape(B * N, D)
    G = _sc_gather(feats2d, idx.reshape(R // IDX_COLS, IDX_COLS))
    centers = feats2d.reshape(B, N, D)[:, ::4, :].reshape(M, D)
    w1a = conv1_w[:, :D]
    w1d = conv1_w[:, D:] - w1a
    out = _conv_pipeline(G, centers, w1a.T, w1d.T, conv2_w.T,
                         bn1_g, bn1_b, bn2_g, bn2_b)
    h = out.reshape(B, S, C).transpose(0, 2, 1)
    return (coords, h)
